# baseline (speedup 1.0000x reference)
import functools
import math

import jax
import jax.numpy as jnp
import numpy as np
from jax import lax
from jax.experimental import pallas as pl
from jax.experimental.pallas import tpu as pltpu
from jax.experimental.pallas import tpu_sc as plsc

H = 10
DH = 12
HD = H * DH
T = 4
G = 16
QROW = 128
KVROW = 256
GROW = 144
NH16 = 16

_EDGE_BLK = 3200
_CS = 64
_C = 128


def _head_sum_mat():
    j = jnp.arange(QROW)[:, None]
    h = jnp.arange(NH16)[None, :]
    return ((j // DH == h) & (j < HD)).astype(jnp.float32)


def _head_bcast_mat():
    h = jnp.arange(NH16)[:, None]
    j = jnp.arange(HD)[None, :]
    return (j // DH == h).astype(jnp.float32)


def _qkv_body(h_ref, wq_ref, wkv_ref, q_ref, kv_ref):
    h = h_ref[...]
    q_ref[...] = jnp.dot(h, wq_ref[...], preferred_element_type=jnp.float32)
    kv_ref[...] = jnp.dot(h, wkv_ref[...], preferred_element_type=jnp.float32)


def _qkv(h, wq_p, wkv_p):
    n = h.shape[0]
    return pl.pallas_call(
        _qkv_body,
        out_shape=[
            jax.ShapeDtypeStruct((n, QROW), jnp.float32),
            jax.ShapeDtypeStruct((n, KVROW), jnp.float32),
        ],
    )(h, wq_p, wkv_p)


def _sc_gather(q, kv, src, dst):
    e = src.shape[0]
    pw = e // 32
    k = pw // _C
    t = pw - k * _C
    tb = max(t, 8)
    mesh = plsc.VectorSubcoreMesh(core_axis_name="c", subcore_axis_name="s", num_cores=2, num_subcores=16)

    assert k % 2 == 0, "double-buffered gather needs an even chunk count"

    @functools.partial(
        pl.kernel,
        out_type=[
            jax.ShapeDtypeStruct((e, QROW), jnp.float32),
            jax.ShapeDtypeStruct((e, KVROW), jnp.float32),
        ],
        mesh=mesh,
        compiler_params=pltpu.CompilerParams(use_tc_tiling_on_sc=False),
        scratch_types=[
            pltpu.VMEM((_C,), jnp.int32),
            pltpu.VMEM((_C,), jnp.int32),
            pltpu.VMEM((_C, QROW), jnp.float32),
            pltpu.VMEM((_C, KVROW), jnp.float32),
            pltpu.VMEM((_C,), jnp.int32),
            pltpu.VMEM((_C,), jnp.int32),
            pltpu.VMEM((_C, QROW), jnp.float32),
            pltpu.VMEM((_C, KVROW), jnp.float32),
            pltpu.VMEM((tb,), jnp.int32),
            pltpu.VMEM((tb,), jnp.int32),
            pltpu.VMEM((tb, QROW), jnp.float32),
            pltpu.VMEM((tb, KVROW), jnp.float32),
            pltpu.SemaphoreType.DMA,
            pltpu.SemaphoreType.DMA,
            pltpu.SemaphoreType.DMA,
            pltpu.SemaphoreType.DMA,
            pltpu.SemaphoreType.DMA,
            pltpu.SemaphoreType.DMA,
            pltpu.SemaphoreType.DMA,
            pltpu.SemaphoreType.DMA,
        ],
    )
    def gk(q_hbm, kv_hbm, src_hbm, dst_hbm, qd_hbm, kvg_hbm,
           di0, si0, qr0, kvr0, di1, si1, qr1, kvr1,
           dit, sit, qrowst, kvrowst,
           gq0, gk0, gq1, gk1, wq0, wk0, wq1, wk1):
        wid = lax.axis_index("s") * 2 + lax.axis_index("c")
        base0 = wid * pw
        bufs = ((di0, si0, qr0, kvr0, gq0, gk0, wq0, wk0),
                (di1, si1, qr1, kvr1, gq1, gk1, wq1, wk1))

        def start(c, p):
            di, si, qr, kvr, gq, gkv, _, _2 = bufs[p]
            b = base0 + c * _C
            pltpu.sync_copy(dst_hbm.at[pl.ds(b, _C)], di)
            pltpu.sync_copy(src_hbm.at[pl.ds(b, _C)], si)
            pltpu.async_copy(q_hbm.at[di], qr, gq)
            pltpu.async_copy(kv_hbm.at[si], kvr, gkv)

        def wait_gather(p):
            di, si, qr, kvr, gq, gkv, _, _2 = bufs[p]
            pltpu.make_async_copy(q_hbm.at[di], qr, gq).wait()
            pltpu.make_async_copy(kv_hbm.at[si], kvr, gkv).wait()

        def writeback(c, p):
            _, _2, qr, kvr, _3, _4, wq, wkv = bufs[p]
            b = base0 + c * _C
            pltpu.async_copy(qr, qd_hbm.at[pl.ds(b, _C)], wq)
            pltpu.async_copy(kvr, kvg_hbm.at[pl.ds(b, _C)], wkv)
            pltpu.make_async_copy(qr, qd_hbm.at[pl.ds(b, _C)], wq).wait()
            pltpu.make_async_copy(kvr, kvg_hbm.at[pl.ds(b, _C)], wkv).wait()

        start(0, 0)

        @pl.loop(0, k // 2)
        def _(i):
            start(2 * i + 1, 1)
            wait_gather(0)
            writeback(2 * i, 0)

            @pl.when(i < k // 2 - 1)
            def _():
                start(2 * i + 2, 0)

            wait_gather(1)
            writeback(2 * i + 1, 1)

        if t > 0:
            b = base0 + k * _C
            pltpu.sync_copy(dst_hbm.at[pl.ds(b, t)], dit)
            pltpu.sync_copy(src_hbm.at[pl.ds(b, t)], sit)
            cp1 = pltpu.async_copy(q_hbm.at[dit], qrowst, gq0)
            cp2 = pltpu.async_copy(kv_hbm.at[sit], kvrowst, gk0)
            cp1.wait()
            cp2.wait()
            pltpu.sync_copy(qrowst, qd_hbm.at[pl.ds(b, t)])
            pltpu.sync_copy(kvrowst, kvg_hbm.at[pl.ds(b, t)])

    return gk(q, kv, src, dst)


def _sc_scatter(wvex, dst, n):
    e = wvex.shape[0]
    pw = e // 32
    k = pw // _CS
    t = pw - k * _CS
    tb = max(t, 8)
    assert k % 2 == 0, "double-buffered scatter needs an even chunk count"
    np_ = ((n + 127) // 128) * 128
    rpt = np_ // 16
    zc = 8
    mesh = plsc.VectorSubcoreMesh(core_axis_name="c", subcore_axis_name="s", num_cores=2, num_subcores=16)

    @functools.partial(
        pl.kernel,
        out_type=jax.ShapeDtypeStruct((2, np_, GROW), jnp.float32),
        mesh=mesh,
        compiler_params=pltpu.CompilerParams(use_tc_tiling_on_sc=False),
        scratch_types=[
            pltpu.VMEM((_CS,), jnp.int32),
            pltpu.VMEM((_CS, GROW), jnp.float32),
            pltpu.VMEM((_CS,), jnp.int32),
            pltpu.VMEM((_CS, GROW), jnp.float32),
            pltpu.VMEM((tb,), jnp.int32),
            pltpu.VMEM((tb, GROW), jnp.float32),
            pltpu.VMEM((zc, GROW), jnp.float32),
            pltpu.VMEM_SHARED((np_, GROW), jnp.float32),
            pltpu.SemaphoreType.DMA,
            pltpu.SemaphoreType.DMA,
            pltpu.SemaphoreType.DMA,
            pltpu.SemaphoreType.DMA,
        ],
    )
    def sk(wvex_hbm, dst_hbm, out_hbm, di0, rows0, di1, rows1,
           dit, rowst, zb, acc_sh, sd0, sr0, sd1, sr1):
        cid = lax.axis_index("c")
        sid = lax.axis_index("s")
        wid = sid * 2 + cid
        base0 = wid * pw

        zv = jnp.zeros((16,), jnp.float32)
        for i in range(zc):
            for j in range(GROW // 16):
                zb[i, pl.ds(j * 16, 16)] = zv
        r0 = sid * rpt

        @pl.loop(0, rpt // zc)
        def _(i):
            pltpu.sync_copy(zb, acc_sh.at[pl.ds(r0 + i * zc, zc)])

        plsc.subcore_barrier()

        sbufs = ((di0, rows0, sd0, sr0), (di1, rows1, sd1, sr1))

        def sload(c, p):
            di, rows, sd, sr = sbufs[p]
            b = base0 + c * _CS
            pltpu.async_copy(dst_hbm.at[pl.ds(b, _CS)], di, sd)
            pltpu.async_copy(wvex_hbm.at[pl.ds(b, _CS)], rows, sr)

        def swait_scat(c, p):
            di, rows, sd, sr = sbufs[p]
            b = base0 + c * _CS
            pltpu.make_async_copy(dst_hbm.at[pl.ds(b, _CS)], di, sd).wait()
            pltpu.make_async_copy(wvex_hbm.at[pl.ds(b, _CS)], rows, sr).wait()
            pltpu.sync_copy(rows, acc_sh.at[di], add=True)

        sload(0, 0)

        @pl.loop(0, k // 2)
        def _(i):
            sload(2 * i + 1, 1)
            swait_scat(2 * i, 0)

            @pl.when(i < k // 2 - 1)
            def _():
                sload(2 * i + 2, 0)

            swait_scat(2 * i + 1, 1)

        if t > 0:
            b = base0 + k * _CS
            pltpu.sync_copy(dst_hbm.at[pl.ds(b, t)], dit)
            pltpu.sync_copy(wvex_hbm.at[pl.ds(b, t)], rowst)
            pltpu.sync_copy(rowst, acc_sh.at[dit], add=True)

        plsc.subcore_barrier()
        pltpu.sync_copy(acc_sh.at[pl.ds(r0, rpt)],
                        out_hbm.at[cid].at[pl.ds(r0, rpt)])

    return sk(wvex, dst)


def _edge_body(qd_ref, kvg_ref, et16_ref, ebl_ref, hs_ref, hb_ref, wvex_ref):
    qd = qd_ref[...]
    ks = kvg_ref[:, :QROW]
    vs = kvg_ref[:, QROW:QROW + HD]
    prod = qd * ks
    scores = jnp.dot(prod, hs_ref[...], preferred_element_type=jnp.float32)
    scores = scores * (1.0 / math.sqrt(DH))
    et16 = et16_ref[...]
    ebl = ebl_ref[...]
    ebias = jnp.zeros_like(scores)
    for t in range(T):
        ebias += jnp.where(et16 == t, ebl[t][None, :], 0.0)
    scores = jnp.clip(scores + ebias, -5.0, 5.0)
    ex = jnp.exp(scores)
    exb = jnp.dot(ex, hb_ref[...], preferred_element_type=jnp.float32)
    blk = qd.shape[0]
    wvex_ref[...] = jnp.concatenate(
        [vs * exb, ex[:, :H], jnp.ones((blk, 1), jnp.float32),
         jnp.zeros((blk, GROW - HD - H - 1), jnp.float32)],
        axis=1)


def _edge_dense(qd, kvg, et16, ebl, hs, hb):
    e = qd.shape[0]
    blk = _EDGE_BLK if e % _EDGE_BLK == 0 else e
    nb = e // blk
    return pl.pallas_call(
        _edge_body,
        grid=(nb,),
        in_specs=[
            pl.BlockSpec((blk, QROW), lambda i: (i, 0)),
            pl.BlockSpec((blk, KVROW), lambda i: (i, 0)),
            pl.BlockSpec((blk, NH16), lambda i: (i, 0)),
            pl.BlockSpec((T, NH16), lambda i: (0, 0)),
            pl.BlockSpec((QROW, NH16), lambda i: (0, 0)),
            pl.BlockSpec((NH16, HD), lambda i: (0, 0)),
        ],
        out_specs=pl.BlockSpec((blk, GROW), lambda i: (i, 0)),
        out_shape=jax.ShapeDtypeStruct((e, GROW), jnp.float32),
    )(qd, kvg, et16, ebl, hs, hb)


def _post_body(h_ref, acc_ref, hb_ref, wo_ref, g1_ref, b1_ref, wf1_ref,
               bf1_ref, wf2_ref, bf2_ref, g2_ref, b2_ref, out_ref):
    nn = h_ref.shape[0]
    wv = acc_ref[0, :nn, :HD] + acc_ref[1, :nn, :HD]
    den = acc_ref[0, :nn, HD:HD + H] + acc_ref[1, :nn, HD:HD + H]
    denb = jnp.dot(den, hb_ref[...], preferred_element_type=jnp.float32)
    agg = wv / (denb + 1e-9)
    h = h_ref[...]
    ho = h + jnp.dot(agg, wo_ref[...], preferred_element_type=jnp.float32)
    mu = jnp.mean(ho, axis=0, keepdims=True)
    var = jnp.mean((ho - mu) ** 2, axis=0, keepdims=True)
    h1 = (ho - mu) * jax.lax.rsqrt(var + 1e-5) * g1_ref[...] + b1_ref[...]
    ff = jnp.maximum(
        jnp.dot(h1, wf1_ref[...], preferred_element_type=jnp.float32)
        + bf1_ref[...], 0.0)
    ff = jnp.dot(ff, wf2_ref[...], preferred_element_type=jnp.float32) + bf2_ref[...]
    h2 = h1 + ff
    mu2 = jnp.mean(h2, axis=0, keepdims=True)
    var2 = jnp.mean((h2 - mu2) ** 2, axis=0, keepdims=True)
    out_ref[...] = (h2 - mu2) * jax.lax.rsqrt(var2 + 1e-5) * g2_ref[...] + b2_ref[...]


def _post(h, acc2, hb, wo, g1, b1, wf1, bf1, wf2, bf2, g2, b2):
    n, d = h.shape
    return pl.pallas_call(
        _post_body,
        out_shape=jax.ShapeDtypeStruct((n, d), jnp.float32),
    )(h, acc2, hb, wo, g1[None, :], b1[None, :], wf1, bf1[None, :],
      wf2, bf2[None, :], g2[None, :], b2[None, :])


def _prescore_body(h_ref, ws_ref, deg_ref, g_ref, norm_ref, hnn_ref):
    hw = jnp.dot(h_ref[...], ws_ref[...], preferred_element_type=jnp.float32)
    nrm = jax.lax.rsqrt(deg_ref[...] + 1.0)
    g_ref[...] = hw * nrm
    norm_ref[...] = nrm
    hnn_ref[...] = hw * nrm * nrm


def _prescore(h, wscore, deg):
    n = h.shape[0]
    return pl.pallas_call(
        _prescore_body,
        out_shape=[jax.ShapeDtypeStruct((n, 1), jnp.float32)] * 3,
    )(h, wscore, deg[:, None])


def _sc_score(g, src, dst):
    n = g.shape[0]
    e = src.shape[0]
    pw = e // 32
    k = pw // _C
    t = pw - k * _C
    tb = max(t, 16)
    mesh = plsc.VectorSubcoreMesh(core_axis_name="c", subcore_axis_name="s", num_cores=2, num_subcores=16)

    @functools.partial(
        pl.kernel,
        out_type=jax.ShapeDtypeStruct((32, n), jnp.float32),
        mesh=mesh,
        compiler_params=pltpu.CompilerParams(needs_layout_passes=False),
        scratch_types=[
            pltpu.VMEM((n,), jnp.float32),
            pltpu.VMEM((n,), jnp.float32),
            pltpu.VMEM((_C,), jnp.int32),
            pltpu.VMEM((_C,), jnp.int32),
            pltpu.VMEM((tb,), jnp.int32),
            pltpu.VMEM((tb,), jnp.int32),
        ],
    )
    def sck(g_hbm, src_hbm, dst_hbm, out_hbm,
            g_v, acc, si, di, sit, dit):
        wid = lax.axis_index("s") * 2 + lax.axis_index("c")
        base0 = wid * pw
        pltpu.sync_copy(g_hbm, g_v)
        zv = jnp.zeros((16,), jnp.float32)

        @pl.loop(0, n // 16)
        def _(i):
            acc[pl.ds(i * 16, 16)] = zv

        def do16(sref, dref, j):
            sj = sref[pl.ds(j * 16, 16)]
            dj = dref[pl.ds(j * 16, 16)]
            gv = plsc.load_gather(g_v, [sj])
            plsc.addupdate_scatter(acc, [dj], gv)

        @pl.loop(0, k)
        def _(i):
            b = base0 + i * _C
            pltpu.sync_copy(src_hbm.at[pl.ds(b, _C)], si)
            pltpu.sync_copy(dst_hbm.at[pl.ds(b, _C)], di)
            for j in range(_C // 16):
                do16(si, di, j)

        if t > 0:
            b = base0 + k * _C
            pltpu.sync_copy(src_hbm.at[pl.ds(b, t)], sit)
            pltpu.sync_copy(dst_hbm.at[pl.ds(b, t)], dit)
            for j in range(t // 16):
                do16(sit, dit, j)

        pltpu.sync_copy(acc, out_hbm.at[wid])

    return sck(g, src, dst)


def _fin_body(kkeep, smat_ref, scol_ref, seg_ref, hp_ref, wm1_ref, bm1_ref,
              wm2_ref, bm2_ref, wm3_ref, bm3_ref, out_ref):
    smat = smat_ref[...]
    bmat = jax.lax.bitcast_convert_type(smat, jnp.int32)
    kmat = bmat ^ (jnp.right_shift(bmat, 31) & jnp.int32(0x7FFFFFFF))

    def cnt_ge(c):
        return jnp.sum((kmat >= c).astype(jnp.int32))

    cnt0 = cnt_ge(jnp.int32(0))
    base = jnp.where(cnt0 >= kkeep, jnp.int32(0), jnp.int32(-2147483648))

    def s1(i, t):
        cand = t + (jnp.int32(1) << (30 - i))
        return jnp.where(cnt_ge(cand) >= kkeep, cand, t)

    thr = jax.lax.fori_loop(0, 31, s1, base)

    eq = (kmat == thr)
    cnt_gt = jnp.sum((kmat > thr).astype(jnp.int32))
    need_eq = kkeep - cnt_gt
    rr, cc = smat.shape
    iota_mat = (jax.lax.broadcasted_iota(jnp.int32, (rr, cc), 0) * cc
                + jax.lax.broadcasted_iota(jnp.int32, (rr, cc), 1))

    def s2(i, t):
        cand = t + (jnp.int32(1) << (13 - i))
        cnte = jnp.sum((eq & (iota_mat < cand)).astype(jnp.int32))
        return jnp.where(cnte <= need_eq, cand, t)

    istar = jax.lax.fori_loop(0, 14, s2, jnp.int32(0))

    scol = scol_ref[...]
    bcol = jax.lax.bitcast_convert_type(scol, jnp.int32)
    kcol = bcol ^ (jnp.right_shift(bcol, 31) & jnp.int32(0x7FFFFFFF))
    npad = scol.shape[0]
    iota_col = jax.lax.broadcasted_iota(jnp.int32, (npad, 1), 0)
    sel = (kcol > thr) | (eq_col := (kcol == thr) & (iota_col < istar))
    self32 = sel.astype(jnp.float32)
    wcol = jnp.tanh(scol) * self32

    hp = hp_ref[...]
    hw = hp * wcol
    seg = seg_ref[...]
    onehot = (seg == jax.lax.broadcasted_iota(jnp.int32, (1, G), 1))
    af = onehot.astype(jnp.float32) * self32
    counts = jax.lax.dot_general(
        af, jnp.ones((npad, 1), jnp.float32), (((0,), (0,)), ((), ())),
        preferred_element_type=jnp.float32)
    sums = jax.lax.dot_general(
        af, hw, (((0,), (0,)), ((), ())),
        preferred_element_type=jnp.float32)
    avg = sums / jnp.maximum(counts, 1.0)

    iota_g = jax.lax.broadcasted_iota(jnp.int32, (G, 1), 0)

    def mxbody(g, mx):
        mg_sel = (seg == g) & sel
        big = jnp.where(mg_sel, hw, -3.0e38)
        m_g = jnp.max(big, axis=0, keepdims=True)
        cnt_g = jnp.sum(mg_sel.astype(jnp.float32))
        m_g = jnp.where(cnt_g > 0, m_g, 0.0)
        return jnp.where(iota_g == g, m_g, mx)

    mx = jax.lax.fori_loop(0, G, mxbody, jnp.zeros((G, hp.shape[1]), jnp.float32))
    o = jnp.concatenate([avg, mx], axis=1)
    o = jnp.maximum(
        jnp.dot(o, wm1_ref[...], preferred_element_type=jnp.float32)
        + bm1_ref[...], 0.0)
    o = jnp.maximum(
        jnp.dot(o, wm2_ref[...], preferred_element_type=jnp.float32)
        + bm2_ref[...], 0.0)
    lg = jnp.dot(o, wm3_ref[...], preferred_element_type=jnp.float32) + bm3_ref[...]
    lmax = jnp.max(lg, axis=1, keepdims=True)
    elg = jnp.exp(lg - lmax)
    out_ref[...] = elg / jnp.sum(elg, axis=1, keepdims=True)


def _final(score, h, segment_ids, wm1, bm1, wm2, bm2, wm3, bm3):
    n, d = h.shape
    np2 = ((n + 127) // 128) * 128
    kkeep = n // 2
    score_p = jnp.concatenate(
        [score, jnp.full((np2 - n,), -jnp.inf, jnp.float32)])
    smat = score_p.reshape(np2 // 128, 128)
    scol = score_p.reshape(np2, 1)
    seg_p = jnp.concatenate(
        [segment_ids.astype(jnp.int32), jnp.full((np2 - n,), G, jnp.int32)])
    seg_col = seg_p.reshape(np2, 1)
    h_p = jnp.concatenate([h, jnp.zeros((np2 - n, d), jnp.float32)], axis=0)
    return pl.pallas_call(
        functools.partial(_fin_body, kkeep),
        out_shape=jax.ShapeDtypeStruct((G, 2), jnp.float32),
    )(smat, scol, seg_col, h_p, wm1, bm1[None, :], wm2, bm2[None, :],
      wm3, bm3[None, :])


def kernel(x, Wq, Wk, Wv, Eb, Wo, bn1_g, bn1_b, Wff1, bff1, Wff2, bff2,
           bn2_g, bn2_b, Wscore, Wm1, bm1, Wm2, bm2, Wm3, bm3,
           edge_index, edge_types, segment_ids):
    n = x.shape[0]
    e = edge_index.shape[1]
    L = Wq.shape[0]
    D = x.shape[1]
    src = edge_index[0]
    dst = edge_index[1]

    hs = _head_sum_mat()
    hb = _head_bcast_mat()
    eb_pad = jnp.pad(Eb, ((0, 0), (0, 0), (0, NH16 - H)))
    et16 = jnp.broadcast_to(edge_types[:, None], (e, NH16)).astype(jnp.int32)
    zpad = jnp.zeros((D, QROW - HD), jnp.float32)

    h = x
    acc_l0 = None
    for l in range(L):
        wq_p = jnp.concatenate([Wq[l], zpad], axis=1)
        wkv_p = jnp.concatenate([Wk[l], zpad, Wv[l], zpad], axis=1)
        q, kv = _qkv(h, wq_p, wkv_p)
        qd, kvg = _sc_gather(q, kv, src, dst)
        wvex = _edge_dense(qd, kvg, et16, eb_pad[l], hs, hb)
        acc2 = _sc_scatter(wvex, dst, n)
        if l == 0:
            acc_l0 = acc2
        h = _post(h, acc2, hb[:H], Wo[l], bn1_g[l], bn1_b[l], Wff1[l], bff1[l],
                  Wff2[l], bff2[l], bn2_g[l], bn2_b[l])

    deg = acc_l0[0, :n, HD + H] + acc_l0[1, :n, HD + H]
    g, norm, hnn = _prescore(h, Wscore, deg)
    parts = _sc_score(g.reshape(n), src, dst)
    score = norm[:, 0] * jnp.sum(parts, axis=0) + hnn[:, 0]

    return _final(score, h, segment_ids, Wm1, bm1, Wm2, bm2, Wm3, bm3)


# speedup vs baseline: 1.2640x; 1.2640x over previous
import functools
import math

import jax
import jax.numpy as jnp
import numpy as np
from jax import lax
from jax.experimental import pallas as pl
from jax.experimental.pallas import tpu as pltpu
from jax.experimental.pallas import tpu_sc as plsc

H = 10
DH = 12
HD = H * DH
T = 4
G = 16
QROW = 128
KVROW = 256
GROW = 144
NH16 = 16

_EDGE_BLK = 3200
_CS = 64
_C = 128


def _head_sum_mat():
    j = jnp.arange(QROW)[:, None]
    h = jnp.arange(NH16)[None, :]
    return ((j // DH == h) & (j < HD)).astype(jnp.float32)


def _head_bcast_mat():
    h = jnp.arange(NH16)[:, None]
    j = jnp.arange(HD)[None, :]
    return (j // DH == h).astype(jnp.float32)


def _qkv_body(h_ref, wq_ref, wkv_ref, q_ref, kv_ref):
    h = h_ref[...]
    q_ref[...] = jnp.dot(h, wq_ref[...], preferred_element_type=jnp.float32)
    kv_ref[...] = jnp.dot(h, wkv_ref[...], preferred_element_type=jnp.float32)


def _qkv(h, wq_p, wkv_p):
    n = h.shape[0]
    return pl.pallas_call(
        _qkv_body,
        out_shape=[
            jax.ShapeDtypeStruct((n, QROW), jnp.float32),
            jax.ShapeDtypeStruct((n, KVROW), jnp.float32),
        ],
    )(h, wq_p, wkv_p)


def _sc_gather(q, kv, src, dst):
    e = src.shape[0]
    pw = e // 32
    k = pw // _C
    t = pw - k * _C
    tb = max(t, 8)
    mesh = plsc.VectorSubcoreMesh(core_axis_name="c", subcore_axis_name="s", num_cores=2, num_subcores=16)

    assert k % 2 == 0, "double-buffered gather needs an even chunk count"

    @functools.partial(
        pl.kernel,
        out_type=[
            jax.ShapeDtypeStruct((e, QROW), jnp.float32),
            jax.ShapeDtypeStruct((e, KVROW), jnp.float32),
        ],
        mesh=mesh,
        scratch_types=[
            pltpu.VMEM((_C,), jnp.int32),
            pltpu.VMEM((_C,), jnp.int32),
            pltpu.VMEM((_C, QROW), jnp.float32),
            pltpu.VMEM((_C, KVROW), jnp.float32),
            pltpu.VMEM((_C,), jnp.int32),
            pltpu.VMEM((_C,), jnp.int32),
            pltpu.VMEM((_C, QROW), jnp.float32),
            pltpu.VMEM((_C, KVROW), jnp.float32),
            pltpu.VMEM((tb,), jnp.int32),
            pltpu.VMEM((tb,), jnp.int32),
            pltpu.VMEM((tb, QROW), jnp.float32),
            pltpu.VMEM((tb, KVROW), jnp.float32),
            pltpu.SemaphoreType.DMA,
            pltpu.SemaphoreType.DMA,
            pltpu.SemaphoreType.DMA,
            pltpu.SemaphoreType.DMA,
            pltpu.SemaphoreType.DMA,
            pltpu.SemaphoreType.DMA,
            pltpu.SemaphoreType.DMA,
            pltpu.SemaphoreType.DMA,
        ],
    )
    def gk(q_hbm, kv_hbm, src_hbm, dst_hbm, qd_hbm, kvg_hbm,
           di0, si0, qr0, kvr0, di1, si1, qr1, kvr1,
           dit, sit, qrowst, kvrowst,
           gq0, gk0, gq1, gk1, wq0, wk0, wq1, wk1):
        wid = lax.axis_index("s") * 2 + lax.axis_index("c")
        base0 = wid * pw
        bufs = ((di0, si0, qr0, kvr0, gq0, gk0, wq0, wk0),
                (di1, si1, qr1, kvr1, gq1, gk1, wq1, wk1))

        def start(c, p):
            di, si, qr, kvr, gq, gkv, _, _2 = bufs[p]
            b = base0 + c * _C
            pltpu.sync_copy(dst_hbm.at[pl.ds(b, _C)], di)
            pltpu.sync_copy(src_hbm.at[pl.ds(b, _C)], si)
            pltpu.async_copy(q_hbm.at[di], qr, gq)
            pltpu.async_copy(kv_hbm.at[si], kvr, gkv)

        def wait_gather(p):
            di, si, qr, kvr, gq, gkv, _, _2 = bufs[p]
            pltpu.make_async_copy(q_hbm.at[di], qr, gq).wait()
            pltpu.make_async_copy(kv_hbm.at[si], kvr, gkv).wait()

        def writeback(c, p):
            _, _2, qr, kvr, _3, _4, wq, wkv = bufs[p]
            b = base0 + c * _C
            pltpu.async_copy(qr, qd_hbm.at[pl.ds(b, _C)], wq)
            pltpu.async_copy(kvr, kvg_hbm.at[pl.ds(b, _C)], wkv)
            pltpu.make_async_copy(qr, qd_hbm.at[pl.ds(b, _C)], wq).wait()
            pltpu.make_async_copy(kvr, kvg_hbm.at[pl.ds(b, _C)], wkv).wait()

        start(0, 0)

        @pl.loop(0, k // 2)
        def _(i):
            start(2 * i + 1, 1)
            wait_gather(0)
            writeback(2 * i, 0)

            @pl.when(i < k // 2 - 1)
            def _():
                start(2 * i + 2, 0)

            wait_gather(1)
            writeback(2 * i + 1, 1)

        if t > 0:
            b = base0 + k * _C
            pltpu.sync_copy(dst_hbm.at[pl.ds(b, t)], dit)
            pltpu.sync_copy(src_hbm.at[pl.ds(b, t)], sit)
            cp1 = pltpu.async_copy(q_hbm.at[dit], qrowst, gq0)
            cp2 = pltpu.async_copy(kv_hbm.at[sit], kvrowst, gk0)
            cp1.wait()
            cp2.wait()
            pltpu.sync_copy(qrowst, qd_hbm.at[pl.ds(b, t)])
            pltpu.sync_copy(kvrowst, kvg_hbm.at[pl.ds(b, t)])

    return gk(q, kv, src, dst)


def _sc_scatter(wvex, dst, n):
    e = wvex.shape[0]
    pw = e // 32
    k = pw // _CS
    t = pw - k * _CS
    tb = max(t, 8)
    assert k % 2 == 0, "double-buffered scatter needs an even chunk count"
    np_ = ((n + 127) // 128) * 128
    rpt = np_ // 16
    zc = 8
    mesh = plsc.VectorSubcoreMesh(core_axis_name="c", subcore_axis_name="s", num_cores=2, num_subcores=16)

    @functools.partial(
        pl.kernel,
        out_type=jax.ShapeDtypeStruct((2, np_, GROW), jnp.float32),
        mesh=mesh,
        compiler_params=pltpu.CompilerParams(use_tc_tiling_on_sc=False),
        scratch_types=[
            pltpu.VMEM((_CS,), jnp.int32),
            pltpu.VMEM((_CS, GROW), jnp.float32),
            pltpu.VMEM((_CS,), jnp.int32),
            pltpu.VMEM((_CS, GROW), jnp.float32),
            pltpu.VMEM((tb,), jnp.int32),
            pltpu.VMEM((tb, GROW), jnp.float32),
            pltpu.VMEM((zc, GROW), jnp.float32),
            pltpu.VMEM_SHARED((np_, GROW), jnp.float32),
            pltpu.SemaphoreType.DMA,
            pltpu.SemaphoreType.DMA,
            pltpu.SemaphoreType.DMA,
            pltpu.SemaphoreType.DMA,
        ],
    )
    def sk(wvex_hbm, dst_hbm, out_hbm, di0, rows0, di1, rows1,
           dit, rowst, zb, acc_sh, sd0, sr0, sd1, sr1):
        cid = lax.axis_index("c")
        sid = lax.axis_index("s")
        wid = sid * 2 + cid
        base0 = wid * pw

        zv = jnp.zeros((16,), jnp.float32)
        for i in range(zc):
            for j in range(GROW // 16):
                zb[i, pl.ds(j * 16, 16)] = zv
        r0 = sid * rpt

        @pl.loop(0, rpt // zc)
        def _(i):
            pltpu.sync_copy(zb, acc_sh.at[pl.ds(r0 + i * zc, zc)])

        plsc.subcore_barrier()

        sbufs = ((di0, rows0, sd0, sr0), (di1, rows1, sd1, sr1))

        def sload(c, p):
            di, rows, sd, sr = sbufs[p]
            b = base0 + c * _CS
            pltpu.async_copy(dst_hbm.at[pl.ds(b, _CS)], di, sd)
            pltpu.async_copy(wvex_hbm.at[pl.ds(b, _CS)], rows, sr)

        def swait_scat(c, p):
            di, rows, sd, sr = sbufs[p]
            b = base0 + c * _CS
            pltpu.make_async_copy(dst_hbm.at[pl.ds(b, _CS)], di, sd).wait()
            pltpu.make_async_copy(wvex_hbm.at[pl.ds(b, _CS)], rows, sr).wait()
            pltpu.sync_copy(rows, acc_sh.at[di], add=True)

        sload(0, 0)

        @pl.loop(0, k // 2)
        def _(i):
            sload(2 * i + 1, 1)
            swait_scat(2 * i, 0)

            @pl.when(i < k // 2 - 1)
            def _():
                sload(2 * i + 2, 0)

            swait_scat(2 * i + 1, 1)

        if t > 0:
            b = base0 + k * _CS
            pltpu.sync_copy(dst_hbm.at[pl.ds(b, t)], dit)
            pltpu.sync_copy(wvex_hbm.at[pl.ds(b, t)], rowst)
            pltpu.sync_copy(rowst, acc_sh.at[dit], add=True)

        plsc.subcore_barrier()
        pltpu.sync_copy(acc_sh.at[pl.ds(r0, rpt)],
                        out_hbm.at[cid].at[pl.ds(r0, rpt)])

    return sk(wvex, dst)


def _edge_body(qd_ref, kvg_ref, et16_ref, ebl_ref, hs_ref, hb_ref, wvex_ref):
    qd = qd_ref[...]
    ks = kvg_ref[:, :QROW]
    vs = kvg_ref[:, QROW:QROW + HD]
    prod = qd * ks
    scores = jnp.dot(prod, hs_ref[...], preferred_element_type=jnp.float32)
    scores = scores * (1.0 / math.sqrt(DH))
    et16 = et16_ref[...]
    ebl = ebl_ref[...]
    ebias = jnp.zeros_like(scores)
    for t in range(T):
        ebias += jnp.where(et16 == t, ebl[t][None, :], 0.0)
    scores = jnp.clip(scores + ebias, -5.0, 5.0)
    ex = jnp.exp(scores)
    exb = jnp.dot(ex, hb_ref[...], preferred_element_type=jnp.float32)
    blk = qd.shape[0]
    wvex_ref[...] = jnp.concatenate(
        [vs * exb, ex[:, :H], jnp.ones((blk, 1), jnp.float32),
         jnp.zeros((blk, GROW - HD - H - 1), jnp.float32)],
        axis=1)


def _edge_dense(qd, kvg, et16, ebl, hs, hb):
    e = qd.shape[0]
    blk = _EDGE_BLK if e % _EDGE_BLK == 0 else e
    nb = e // blk
    return pl.pallas_call(
        _edge_body,
        grid=(nb,),
        in_specs=[
            pl.BlockSpec((blk, QROW), lambda i: (i, 0)),
            pl.BlockSpec((blk, KVROW), lambda i: (i, 0)),
            pl.BlockSpec((blk, NH16), lambda i: (i, 0)),
            pl.BlockSpec((T, NH16), lambda i: (0, 0)),
            pl.BlockSpec((QROW, NH16), lambda i: (0, 0)),
            pl.BlockSpec((NH16, HD), lambda i: (0, 0)),
        ],
        out_specs=pl.BlockSpec((blk, GROW), lambda i: (i, 0)),
        out_shape=jax.ShapeDtypeStruct((e, GROW), jnp.float32),
    )(qd, kvg, et16, ebl, hs, hb)


def _post_body(h_ref, acc_ref, hb_ref, wo_ref, g1_ref, b1_ref, wf1_ref,
               bf1_ref, wf2_ref, bf2_ref, g2_ref, b2_ref, out_ref):
    nn = h_ref.shape[0]
    wv = acc_ref[0, :nn, :HD] + acc_ref[1, :nn, :HD]
    den = acc_ref[0, :nn, HD:HD + H] + acc_ref[1, :nn, HD:HD + H]
    denb = jnp.dot(den, hb_ref[...], preferred_element_type=jnp.float32)
    agg = wv / (denb + 1e-9)
    h = h_ref[...]
    ho = h + jnp.dot(agg, wo_ref[...], preferred_element_type=jnp.float32)
    mu = jnp.mean(ho, axis=0, keepdims=True)
    var = jnp.mean((ho - mu) ** 2, axis=0, keepdims=True)
    h1 = (ho - mu) * jax.lax.rsqrt(var + 1e-5) * g1_ref[...] + b1_ref[...]
    ff = jnp.maximum(
        jnp.dot(h1, wf1_ref[...], preferred_element_type=jnp.float32)
        + bf1_ref[...], 0.0)
    ff = jnp.dot(ff, wf2_ref[...], preferred_element_type=jnp.float32) + bf2_ref[...]
    h2 = h1 + ff
    mu2 = jnp.mean(h2, axis=0, keepdims=True)
    var2 = jnp.mean((h2 - mu2) ** 2, axis=0, keepdims=True)
    out_ref[...] = (h2 - mu2) * jax.lax.rsqrt(var2 + 1e-5) * g2_ref[...] + b2_ref[...]


def _post(h, acc2, hb, wo, g1, b1, wf1, bf1, wf2, bf2, g2, b2):
    n, d = h.shape
    return pl.pallas_call(
        _post_body,
        out_shape=jax.ShapeDtypeStruct((n, d), jnp.float32),
    )(h, acc2, hb, wo, g1[None, :], b1[None, :], wf1, bf1[None, :],
      wf2, bf2[None, :], g2[None, :], b2[None, :])


def _prescore_body(h_ref, ws_ref, deg_ref, g_ref, norm_ref, hnn_ref):
    hw = jnp.dot(h_ref[...], ws_ref[...], preferred_element_type=jnp.float32)
    nrm = jax.lax.rsqrt(deg_ref[...] + 1.0)
    g_ref[...] = hw * nrm
    norm_ref[...] = nrm
    hnn_ref[...] = hw * nrm * nrm


def _prescore(h, wscore, deg):
    n = h.shape[0]
    return pl.pallas_call(
        _prescore_body,
        out_shape=[jax.ShapeDtypeStruct((n, 1), jnp.float32)] * 3,
    )(h, wscore, deg[:, None])


def _sc_score(g, src, dst):
    n = g.shape[0]
    e = src.shape[0]
    pw = e // 32
    k = pw // _C
    t = pw - k * _C
    tb = max(t, 16)
    mesh = plsc.VectorSubcoreMesh(core_axis_name="c", subcore_axis_name="s", num_cores=2, num_subcores=16)

    @functools.partial(
        pl.kernel,
        out_type=jax.ShapeDtypeStruct((32, n), jnp.float32),
        mesh=mesh,
        compiler_params=pltpu.CompilerParams(needs_layout_passes=False),
        scratch_types=[
            pltpu.VMEM((n,), jnp.float32),
            pltpu.VMEM((n,), jnp.float32),
            pltpu.VMEM((_C,), jnp.int32),
            pltpu.VMEM((_C,), jnp.int32),
            pltpu.VMEM((tb,), jnp.int32),
            pltpu.VMEM((tb,), jnp.int32),
        ],
    )
    def sck(g_hbm, src_hbm, dst_hbm, out_hbm,
            g_v, acc, si, di, sit, dit):
        wid = lax.axis_index("s") * 2 + lax.axis_index("c")
        base0 = wid * pw
        pltpu.sync_copy(g_hbm, g_v)
        zv = jnp.zeros((16,), jnp.float32)

        @pl.loop(0, n // 16)
        def _(i):
            acc[pl.ds(i * 16, 16)] = zv

        def do16(sref, dref, j):
            sj = sref[pl.ds(j * 16, 16)]
            dj = dref[pl.ds(j * 16, 16)]
            gv = plsc.load_gather(g_v, [sj])
            plsc.addupdate_scatter(acc, [dj], gv)

        @pl.loop(0, k)
        def _(i):
            b = base0 + i * _C
            pltpu.sync_copy(src_hbm.at[pl.ds(b, _C)], si)
            pltpu.sync_copy(dst_hbm.at[pl.ds(b, _C)], di)
            for j in range(_C // 16):
                do16(si, di, j)

        if t > 0:
            b = base0 + k * _C
            pltpu.sync_copy(src_hbm.at[pl.ds(b, t)], sit)
            pltpu.sync_copy(dst_hbm.at[pl.ds(b, t)], dit)
            for j in range(t // 16):
                do16(sit, dit, j)

        pltpu.sync_copy(acc, out_hbm.at[wid])

    return sck(g, src, dst)


def _fin_body(kkeep, smat_ref, scol_ref, seg_ref, hp_ref, wm1_ref, bm1_ref,
              wm2_ref, bm2_ref, wm3_ref, bm3_ref, out_ref):
    smat = smat_ref[...]
    bmat = jax.lax.bitcast_convert_type(smat, jnp.int32)
    kmat = bmat ^ (jnp.right_shift(bmat, 31) & jnp.int32(0x7FFFFFFF))

    def cnt_ge(c):
        return jnp.sum((kmat >= c).astype(jnp.int32))

    cnt0 = cnt_ge(jnp.int32(0))
    base = jnp.where(cnt0 >= kkeep, jnp.int32(0), jnp.int32(-2147483648))

    def s1(i, t):
        cand = t + (jnp.int32(1) << (30 - i))
        return jnp.where(cnt_ge(cand) >= kkeep, cand, t)

    thr = jax.lax.fori_loop(0, 31, s1, base)

    eq = (kmat == thr)
    cnt_gt = jnp.sum((kmat > thr).astype(jnp.int32))
    need_eq = kkeep - cnt_gt
    rr, cc = smat.shape
    iota_mat = (jax.lax.broadcasted_iota(jnp.int32, (rr, cc), 0) * cc
                + jax.lax.broadcasted_iota(jnp.int32, (rr, cc), 1))

    def s2(i, t):
        cand = t + (jnp.int32(1) << (13 - i))
        cnte = jnp.sum((eq & (iota_mat < cand)).astype(jnp.int32))
        return jnp.where(cnte <= need_eq, cand, t)

    istar = jax.lax.fori_loop(0, 14, s2, jnp.int32(0))

    scol = scol_ref[...]
    bcol = jax.lax.bitcast_convert_type(scol, jnp.int32)
    kcol = bcol ^ (jnp.right_shift(bcol, 31) & jnp.int32(0x7FFFFFFF))
    npad = scol.shape[0]
    iota_col = jax.lax.broadcasted_iota(jnp.int32, (npad, 1), 0)
    sel = (kcol > thr) | (eq_col := (kcol == thr) & (iota_col < istar))
    self32 = sel.astype(jnp.float32)
    wcol = jnp.tanh(scol) * self32

    hp = hp_ref[...]
    hw = hp * wcol
    seg = seg_ref[...]
    onehot = (seg == jax.lax.broadcasted_iota(jnp.int32, (1, G), 1))
    af = onehot.astype(jnp.float32) * self32
    counts = jax.lax.dot_general(
        af, jnp.ones((npad, 1), jnp.float32), (((0,), (0,)), ((), ())),
        preferred_element_type=jnp.float32)
    sums = jax.lax.dot_general(
        af, hw, (((0,), (0,)), ((), ())),
        preferred_element_type=jnp.float32)
    avg = sums / jnp.maximum(counts, 1.0)

    iota_g = jax.lax.broadcasted_iota(jnp.int32, (G, 1), 0)

    def mxbody(g, mx):
        mg_sel = (seg == g) & sel
        big = jnp.where(mg_sel, hw, -3.0e38)
        m_g = jnp.max(big, axis=0, keepdims=True)
        cnt_g = jnp.sum(mg_sel.astype(jnp.float32))
        m_g = jnp.where(cnt_g > 0, m_g, 0.0)
        return jnp.where(iota_g == g, m_g, mx)

    mx = jax.lax.fori_loop(0, G, mxbody, jnp.zeros((G, hp.shape[1]), jnp.float32))
    o = jnp.concatenate([avg, mx], axis=1)
    o = jnp.maximum(
        jnp.dot(o, wm1_ref[...], preferred_element_type=jnp.float32)
        + bm1_ref[...], 0.0)
    o = jnp.maximum(
        jnp.dot(o, wm2_ref[...], preferred_element_type=jnp.float32)
        + bm2_ref[...], 0.0)
    lg = jnp.dot(o, wm3_ref[...], preferred_element_type=jnp.float32) + bm3_ref[...]
    lmax = jnp.max(lg, axis=1, keepdims=True)
    elg = jnp.exp(lg - lmax)
    out_ref[...] = elg / jnp.sum(elg, axis=1, keepdims=True)


def _final(score, h, segment_ids, wm1, bm1, wm2, bm2, wm3, bm3):
    n, d = h.shape
    np2 = ((n + 127) // 128) * 128
    kkeep = n // 2
    score_p = jnp.concatenate(
        [score, jnp.full((np2 - n,), -jnp.inf, jnp.float32)])
    smat = score_p.reshape(np2 // 128, 128)
    scol = score_p.reshape(np2, 1)
    seg_p = jnp.concatenate(
        [segment_ids.astype(jnp.int32), jnp.full((np2 - n,), G, jnp.int32)])
    seg_col = seg_p.reshape(np2, 1)
    h_p = jnp.concatenate([h, jnp.zeros((np2 - n, d), jnp.float32)], axis=0)
    return pl.pallas_call(
        functools.partial(_fin_body, kkeep),
        out_shape=jax.ShapeDtypeStruct((G, 2), jnp.float32),
    )(smat, scol, seg_col, h_p, wm1, bm1[None, :], wm2, bm2[None, :],
      wm3, bm3[None, :])


def kernel(x, Wq, Wk, Wv, Eb, Wo, bn1_g, bn1_b, Wff1, bff1, Wff2, bff2,
           bn2_g, bn2_b, Wscore, Wm1, bm1, Wm2, bm2, Wm3, bm3,
           edge_index, edge_types, segment_ids):
    n = x.shape[0]
    e = edge_index.shape[1]
    L = Wq.shape[0]
    D = x.shape[1]
    src = edge_index[0]
    dst = edge_index[1]

    hs = _head_sum_mat()
    hb = _head_bcast_mat()
    eb_pad = jnp.pad(Eb, ((0, 0), (0, 0), (0, NH16 - H)))
    et16 = jnp.broadcast_to(edge_types[:, None], (e, NH16)).astype(jnp.int32)
    zpad = jnp.zeros((D, QROW - HD), jnp.float32)

    h = x
    acc_l0 = None
    for l in range(L):
        wq_p = jnp.concatenate([Wq[l], zpad], axis=1)
        wkv_p = jnp.concatenate([Wk[l], zpad, Wv[l], zpad], axis=1)
        q, kv = _qkv(h, wq_p, wkv_p)
        qd, kvg = _sc_gather(q, kv, src, dst)
        wvex = _edge_dense(qd, kvg, et16, eb_pad[l], hs, hb)
        acc2 = _sc_scatter(wvex, dst, n)
        if l == 0:
            acc_l0 = acc2
        h = _post(h, acc2, hb[:H], Wo[l], bn1_g[l], bn1_b[l], Wff1[l], bff1[l],
                  Wff2[l], bff2[l], bn2_g[l], bn2_b[l])

    deg = acc_l0[0, :n, HD + H] + acc_l0[1, :n, HD + H]
    g, norm, hnn = _prescore(h, Wscore, deg)
    parts = _sc_score(g.reshape(n), src, dst)
    score = norm[:, 0] * jnp.sum(parts, axis=0) + hnn[:, 0]

    return _final(score, h, segment_ids, Wm1, bm1, Wm2, bm2, Wm3, bm3)
